# baseline (speedup 1.0000x reference)
import functools

import jax
import jax.numpy as jnp
import numpy as np
from jax import lax
from jax.experimental import pallas as pl
from jax.experimental.pallas import tpu as pltpu
from jax.experimental.pallas import tpu_sc as plsc

_MASKING_RATE = 0.4
_MSK_SCALAR = -1.0

_NC, _NS = 2, 16
_NW = _NC * _NS
_C = 40


def _rotl32(x, r):
    return (x << np.uint32(r)) | (x >> np.uint32(32 - r))


def _threefry2x32(kp, x0, x1):
    ks = [np.uint32(kp[0]), np.uint32(kp[1]),
          np.uint32(kp[0]) ^ np.uint32(kp[1]) ^ np.uint32(0x1BD11BDA)]
    rot = ((13, 15, 26, 6), (17, 29, 16, 24))
    with np.errstate(over="ignore"):
        x0 = x0 + ks[0]
        x1 = x1 + ks[1]
        for i in range(5):
            for r in rot[i % 2]:
                x0 = x0 + x1
                x1 = _rotl32(x1, r)
                x1 = x1 ^ x0
            x0 = x0 + ks[(i + 1) % 3]
            x1 = x1 + ks[(i + 2) % 3] + np.uint32(i + 1)
    return x0, x1


@functools.lru_cache(maxsize=None)
def _mask_bool(num_patches: int) -> np.ndarray:
    k = int(_MASKING_RATE * num_patches)
    f0, f1 = _threefry2x32((np.uint32(0), np.uint32(0)),
                           np.zeros(1, np.uint32), np.ones(1, np.uint32))
    y0, y1 = _threefry2x32((f0[0], f1[0]),
                           np.zeros(num_patches, np.uint32),
                           np.arange(num_patches, dtype=np.uint32))
    bits = y0 ^ y1
    u = ((bits >> np.uint32(9)) | np.uint32(0x3F800000)).view(np.float32) \
        - np.float32(1.0)
    u = np.maximum(np.float32(0.0), u)
    idx = np.argsort(u, kind="stable")[:k]
    masked = np.zeros(num_patches, dtype=bool)
    masked[idx] = True
    return masked


def _pad_split(rows: np.ndarray, nw: int, c: int) -> np.ndarray:
    per_w = -(-len(rows) // nw)
    per_w = -(-per_w // c) * c
    total = per_w * nw
    pad = np.full(total - len(rows), rows[-1], dtype=np.int32)
    return np.concatenate([rows.astype(np.int32), pad]).reshape(nw, per_w // c, c)


@functools.lru_cache(maxsize=None)
def _row_lists(batch: int, num_patches: int):
    masked = _mask_bool(num_patches)
    midx = np.nonzero(masked)[0]
    uidx = np.nonzero(~masked)[0]
    offs = np.arange(batch)[:, None] * num_patches
    fill_rows = (offs + midx[None, :]).ravel()
    copy_rows = (offs + uidx[None, :]).ravel()
    return _pad_split(fill_rows, _NW, _C), _pad_split(copy_rows, _NW, _C)


def kernel(x):
    B, P, D = x.shape
    R = B * P
    fidx_np, cidx_np = _row_lists(B, P)
    NFC = fidx_np.shape[1]
    NCC = cidx_np.shape[1]
    fidx = jnp.asarray(fidx_np)
    cidx = jnp.asarray(cidx_np)
    neg = jnp.full((_C, D), _MSK_SCALAR, dtype=x.dtype)
    x2 = x.reshape(R, D)

    mesh = plsc.VectorSubcoreMesh(core_axis_name="c", subcore_axis_name="s")

    @functools.partial(
        pl.kernel,
        out_type=jax.ShapeDtypeStruct((R, D), x.dtype),
        mesh=mesh,
        scratch_types=[
            pltpu.VMEM((NFC, _C), jnp.int32),
            pltpu.VMEM((NCC, _C), jnp.int32),
            pltpu.VMEM((_C, D), jnp.float32),
            pltpu.VMEM((_C, D), jnp.float32),
            pltpu.VMEM((_C, D), jnp.float32),
            pltpu.SemaphoreType.DMA,
            pltpu.SemaphoreType.DMA,
            pltpu.SemaphoreType.DMA,
        ],
    )
    def sc_fill_copy(x_hbm, fidx_hbm, cidx_hbm, neg_hbm, out_hbm,
                     fidx_v, cidx_v, neg_v, buf_a, buf_b,
                     sem_f, sem_g, sem_s):
        w = lax.axis_index("s") * _NC + lax.axis_index("c")
        pltpu.sync_copy(fidx_hbm.at[w], fidx_v)
        pltpu.sync_copy(cidx_hbm.at[w], cidx_v)
        pltpu.sync_copy(neg_hbm, neg_v)

        fills = []
        bufs = (buf_a, buf_b)
        ga = pltpu.async_copy(x_hbm.at[cidx_v.at[0]], bufs[0], sem_g)
        prev_sc = None
        for i in range(NCC):
            cur = bufs[i % 2]
            nxt = bufs[(i + 1) % 2]
            if i < NFC:
                fills.append(
                    pltpu.async_copy(neg_v, out_hbm.at[fidx_v.at[i]], sem_f))
            if prev_sc is not None:
                prev_sc.wait()
            ga_next = None
            if i + 1 < NCC:
                ga_next = pltpu.async_copy(x_hbm.at[cidx_v.at[i + 1]], nxt, sem_g)
            ga.wait()
            prev_sc = pltpu.async_copy(cur, out_hbm.at[cidx_v.at[i]], sem_s)
            ga = ga_next
        prev_sc.wait()
        for f in fills:
            f.wait()

    out = sc_fill_copy(x2, fidx, cidx, neg)
    return out.reshape(B, P, D)


# speedup vs baseline: 1.8684x; 1.8684x over previous
import functools

import jax
import jax.numpy as jnp
import numpy as np
from jax.experimental import pallas as pl
from jax.experimental.pallas import tpu as pltpu

_MASKING_RATE = 0.4
_MSK_SCALAR = -1.0


def _rotl32(x, r):
    return (x << np.uint32(r)) | (x >> np.uint32(32 - r))


def _threefry2x32(kp, x0, x1):
    ks = [np.uint32(kp[0]), np.uint32(kp[1]),
          np.uint32(kp[0]) ^ np.uint32(kp[1]) ^ np.uint32(0x1BD11BDA)]
    rot = ((13, 15, 26, 6), (17, 29, 16, 24))
    with np.errstate(over="ignore"):
        x0 = x0 + ks[0]
        x1 = x1 + ks[1]
        for i in range(5):
            for r in rot[i % 2]:
                x0 = x0 + x1
                x1 = _rotl32(x1, r)
                x1 = x1 ^ x0
            x0 = x0 + ks[(i + 1) % 3]
            x1 = x1 + ks[(i + 2) % 3] + np.uint32(i + 1)
    return x0, x1


@functools.lru_cache(maxsize=None)
def _mask_bool(num_patches: int) -> np.ndarray:
    k = int(_MASKING_RATE * num_patches)
    f0, f1 = _threefry2x32((np.uint32(0), np.uint32(0)),
                           np.zeros(1, np.uint32), np.ones(1, np.uint32))
    y0, y1 = _threefry2x32((f0[0], f1[0]),
                           np.zeros(num_patches, np.uint32),
                           np.arange(num_patches, dtype=np.uint32))
    bits = y0 ^ y1
    u = ((bits >> np.uint32(9)) | np.uint32(0x3F800000)).view(np.float32) \
        - np.float32(1.0)
    u = np.maximum(np.float32(0.0), u)
    idx = np.argsort(u, kind="stable")[:k]
    masked = np.zeros(num_patches, dtype=bool)
    masked[idx] = True
    return masked


def _select_body(m_ref, x_ref, o_ref, *, lanes):
    m = m_ref[...]
    for k in range(x_ref.shape[1] // lanes):
        sl = slice(k * lanes, (k + 1) * lanes)
        o_ref[:, sl] = jnp.where(m, _MSK_SCALAR, x_ref[:, sl])


def kernel(x):
    B, P, D = x.shape
    R = B * P
    lanes = 128
    masked = np.repeat(
        np.tile(_mask_bool(P), B).reshape(R, 1), lanes, axis=1)
    m = jnp.asarray(masked)
    x2 = x.reshape(R, D)
    PB = 2048
    out = pl.pallas_call(
        functools.partial(_select_body, lanes=lanes),
        grid=(R // PB,),
        in_specs=[
            pl.BlockSpec((PB, lanes), lambda j: (j, 0)),
            pl.BlockSpec((PB, D), lambda j: (j, 0)),
        ],
        out_specs=pl.BlockSpec((PB, D), lambda j: (j, 0)),
        out_shape=jax.ShapeDtypeStruct((R, D), x.dtype),
        compiler_params=pltpu.CompilerParams(
            dimension_semantics=("parallel",)),
    )(m, x2)
    return out.reshape(B, P, D)


# speedup vs baseline: 1.9209x; 1.0280x over previous
import functools

import jax
import jax.numpy as jnp
import numpy as np
from jax.experimental import pallas as pl
from jax.experimental.pallas import tpu as pltpu

_MASKING_RATE = 0.4
_MSK_SCALAR = -1.0


def _rotl32(x, r):
    return (x << np.uint32(r)) | (x >> np.uint32(32 - r))


def _threefry2x32(kp, x0, x1):
    ks = [np.uint32(kp[0]), np.uint32(kp[1]),
          np.uint32(kp[0]) ^ np.uint32(kp[1]) ^ np.uint32(0x1BD11BDA)]
    rot = ((13, 15, 26, 6), (17, 29, 16, 24))
    with np.errstate(over="ignore"):
        x0 = x0 + ks[0]
        x1 = x1 + ks[1]
        for i in range(5):
            for r in rot[i % 2]:
                x0 = x0 + x1
                x1 = _rotl32(x1, r)
                x1 = x1 ^ x0
            x0 = x0 + ks[(i + 1) % 3]
            x1 = x1 + ks[(i + 2) % 3] + np.uint32(i + 1)
    return x0, x1


@functools.lru_cache(maxsize=None)
def _mask_bool(num_patches: int) -> np.ndarray:
    k = int(_MASKING_RATE * num_patches)
    f0, f1 = _threefry2x32((np.uint32(0), np.uint32(0)),
                           np.zeros(1, np.uint32), np.ones(1, np.uint32))
    y0, y1 = _threefry2x32((f0[0], f1[0]),
                           np.zeros(num_patches, np.uint32),
                           np.arange(num_patches, dtype=np.uint32))
    bits = y0 ^ y1
    u = ((bits >> np.uint32(9)) | np.uint32(0x3F800000)).view(np.float32) \
        - np.float32(1.0)
    u = np.maximum(np.float32(0.0), u)
    idx = np.argsort(u, kind="stable")[:k]
    masked = np.zeros(num_patches, dtype=bool)
    masked[idx] = True
    return masked


def _select_body(m_ref, x_ref, o_ref, *, pb):
    j = pl.program_id(0)
    m = m_ref[pl.ds(j * pb, pb), :]
    o_ref[...] = jnp.where(m, _MSK_SCALAR, x_ref[...])


def kernel(x):
    B, P, D = x.shape
    R = B * P
    masked = np.tile(_mask_bool(P), B).reshape(R, 1)
    m = jnp.asarray(masked)
    x2 = x.reshape(R, D)
    PB = 2048
    out = pl.pallas_call(
        functools.partial(_select_body, pb=PB),
        grid=(R // PB,),
        in_specs=[
            pl.BlockSpec((R, 1), lambda j: (0, 0)),
            pl.BlockSpec((PB, D), lambda j: (j, 0)),
        ],
        out_specs=pl.BlockSpec((PB, D), lambda j: (j, 0)),
        out_shape=jax.ShapeDtypeStruct((R, D), x.dtype),
        compiler_params=pltpu.CompilerParams(
            dimension_semantics=("parallel",)),
    )(m, x2)
    return out.reshape(B, P, D)
